# initial kernel scaffold (unmeasured)
import jax
import jax.numpy as jnp
from jax import lax
from jax.experimental import pallas as pl
from jax.experimental.pallas import tpu as pltpu

N_DEV = 32
N_ROWS = 512
D_IN = 256
D_OUT = 512
N_EXP = 64
CAP = 6
EXP_PER_DEV = 2
ROWS_PER_DEV = N_ROWS // N_DEV
SLOTS = EXP_PER_DEV * CAP
PAD = 8


def kernel(x, router_W, route_idx, expert_W):
    del router_W

    def body(x_ref, idx_ref, w_ref, out_ref, xg_ref, y_ref, send_sems, recv_sem):
        me = lax.axis_index("i")

        e = idx_ref[:, :]
        row_iota = lax.broadcasted_iota(jnp.int32, (N_ROWS, 1), 0)

        col = lax.broadcasted_iota(jnp.int32, (N_ROWS, N_EXP), 1)
        onehot = (col == e).astype(jnp.float32)
        tri_r = lax.broadcasted_iota(jnp.int32, (N_ROWS, N_ROWS), 0)
        tri_c = lax.broadcasted_iota(jnp.int32, (N_ROWS, N_ROWS), 1)
        tri = (tri_c <= tri_r).astype(jnp.float32)
        counts = jnp.dot(tri, onehot, preferred_element_type=jnp.float32)
        inc = jnp.sum(counts * onehot, axis=1, keepdims=True)
        keep = inc <= float(CAP)

        xg_ref[:, :] = jnp.zeros((EXP_PER_DEV * PAD, D_IN), jnp.float32)
        slot_valid = []
        slot_row = []
        for e_loc in range(EXP_PER_DEV):
            my_e = me * EXP_PER_DEV + e_loc
            for s in range(CAP):
                pred = (e == my_e) & (inc == float(s + 1))
                hit = jnp.max(pred.astype(jnp.int32)) > 0
                r = jnp.max(jnp.where(pred, row_iota, -1))
                slot_valid.append(hit)
                slot_row.append(r)
                rc = jnp.maximum(r, 0)
                xg_ref[pl.ds(e_loc * PAD + s, 1), :] = x_ref[pl.ds(rc, 1), :]
        for e_loc in range(EXP_PER_DEV):
            y_ref[pl.ds(e_loc * PAD, PAD), :] = jnp.dot(
                xg_ref[pl.ds(e_loc * PAD, PAD), :],
                w_ref[e_loc, :, :],
                preferred_element_type=jnp.float32,
            )

        remote_flags = []
        for k in range(SLOTS):
            e_loc, s = divmod(k, CAP)
            idx = e_loc * PAD + s
            rc = jnp.maximum(slot_row[k], 0)
            dest = rc // ROWS_PER_DEV
            drow = lax.rem(rc, ROWS_PER_DEV)
            remote = slot_valid[k] & (dest != me)
            local = slot_valid[k] & (dest == me)
            remote_flags.append(remote)

            @pl.when(remote)
            def _(idx=idx, k=k, dest=dest, drow=drow):
                rdma = pltpu.make_async_remote_copy(
                    src_ref=y_ref.at[pl.ds(idx, 1)],
                    dst_ref=out_ref.at[pl.ds(drow, 1)],
                    send_sem=send_sems.at[k],
                    recv_sem=recv_sem,
                    device_id=(dest,),
                    device_id_type=pl.DeviceIdType.MESH,
                )
                rdma.start()

            @pl.when(local)
            def _(idx=idx, drow=drow):
                out_ref[pl.ds(drow, 1), :] = y_ref[pl.ds(idx, 1), :]

        expected = jnp.int32(0)
        base = me * ROWS_PER_DEV
        for j in range(ROWS_PER_DEV):
            sel = row_iota == base + j
            kj = jnp.max((sel & keep).astype(jnp.int32)) > 0
            ej = jnp.max(jnp.where(sel, e, 0))
            owner = ej // EXP_PER_DEV
            expected = expected + (kj & (owner != me)).astype(jnp.int32)

            @pl.when(jnp.logical_not(kj))
            def _(j=j):
                out_ref[pl.ds(j, 1), :] = jnp.zeros((1, D_OUT), jnp.float32)

        def wait_one(i, carry):
            dummy = pltpu.make_async_remote_copy(
                src_ref=y_ref.at[pl.ds(0, 1)],
                dst_ref=out_ref.at[pl.ds(0, 1)],
                send_sem=send_sems.at[0],
                recv_sem=recv_sem,
                device_id=(me,),
                device_id_type=pl.DeviceIdType.MESH,
            )
            dummy.wait_recv()
            return carry

        lax.fori_loop(0, expected, wait_one, jnp.int32(0))

        for k in range(SLOTS):
            e_loc, s = divmod(k, CAP)
            idx = e_loc * PAD + s

            @pl.when(remote_flags[k])
            def _(idx=idx, k=k):
                d = pltpu.make_async_remote_copy(
                    src_ref=y_ref.at[pl.ds(idx, 1)],
                    dst_ref=out_ref.at[pl.ds(0, 1)],
                    send_sem=send_sems.at[k],
                    recv_sem=recv_sem,
                    device_id=(me,),
                    device_id_type=pl.DeviceIdType.MESH,
                )
                d.wait_send()

    return pl.pallas_call(
        body,
        out_shape=jax.ShapeDtypeStruct((ROWS_PER_DEV, D_OUT), jnp.float32),
        in_specs=[
            pl.BlockSpec(memory_space=pltpu.VMEM),
            pl.BlockSpec(memory_space=pltpu.VMEM),
            pl.BlockSpec(memory_space=pltpu.VMEM),
        ],
        out_specs=pl.BlockSpec(memory_space=pltpu.VMEM),
        scratch_shapes=[
            pltpu.VMEM((EXP_PER_DEV * PAD, D_IN), jnp.float32),
            pltpu.VMEM((EXP_PER_DEV * PAD, D_OUT), jnp.float32),
            pltpu.SemaphoreType.DMA((SLOTS,)),
            pltpu.SemaphoreType.DMA,
        ],
        compiler_params=pltpu.CompilerParams(collective_id=0),
    )(x, route_idx, expert_W)


# baseline (device time: 24081 ns/iter reference)
import jax
import jax.numpy as jnp
from jax import lax
from jax.experimental import pallas as pl
from jax.experimental.pallas import tpu as pltpu

N_DEV = 32
N_ROWS = 512
D_IN = 256
D_OUT = 512
N_EXP = 64
CAP = 6
EXP_PER_DEV = 2
ROWS_PER_DEV = N_ROWS // N_DEV
SLOTS = EXP_PER_DEV * CAP
PAD = 8


def kernel(x, router_W, route_idx, expert_W):
    del router_W

    def body(x_ref, idx_ref, w_ref, out_ref, xg_ref, y_ref, send_sems, recv_sem):
        me = lax.axis_index("i")

        e = idx_ref[:, :]
        row_iota = lax.broadcasted_iota(jnp.int32, (N_ROWS, 1), 0)

        col = lax.broadcasted_iota(jnp.int32, (N_ROWS, N_EXP), 1)
        onehot = (col == e).astype(jnp.float32)
        tri_r = lax.broadcasted_iota(jnp.int32, (N_ROWS, N_ROWS), 0)
        tri_c = lax.broadcasted_iota(jnp.int32, (N_ROWS, N_ROWS), 1)
        tri = (tri_c <= tri_r).astype(jnp.float32)
        counts = jnp.dot(tri, onehot, preferred_element_type=jnp.float32)
        inc = jnp.sum(counts * onehot, axis=1, keepdims=True)
        keep = inc <= float(CAP)

        xg_ref[:, :] = jnp.zeros((EXP_PER_DEV * PAD, D_IN), jnp.float32)
        slot_valid = []
        slot_row = []
        for e_loc in range(EXP_PER_DEV):
            my_e = me * EXP_PER_DEV + e_loc
            for s in range(CAP):
                pred = (e == my_e) & (inc == float(s + 1))
                hit = jnp.max(pred.astype(jnp.int32)) > 0
                r = jnp.max(jnp.where(pred, row_iota, -1))
                slot_valid.append(hit)
                slot_row.append(r)
                rc = jnp.maximum(r, 0)
                xg_ref[pl.ds(e_loc * PAD + s, 1), :] = x_ref[pl.ds(rc, 1), :]
        for e_loc in range(EXP_PER_DEV):
            y_ref[pl.ds(e_loc * PAD, PAD), :] = jnp.dot(
                xg_ref[pl.ds(e_loc * PAD, PAD), :],
                w_ref[e_loc, :, :],
                preferred_element_type=jnp.float32,
            )

        remote_flags = []
        for k in range(SLOTS):
            e_loc, s = divmod(k, CAP)
            idx = e_loc * PAD + s
            rc = jnp.maximum(slot_row[k], 0)
            dest = rc // ROWS_PER_DEV
            drow = lax.rem(rc, ROWS_PER_DEV)
            remote = slot_valid[k] & (dest != me)
            local = slot_valid[k] & (dest == me)
            remote_flags.append(remote)

            @pl.when(remote)
            def _(idx=idx, k=k, dest=dest, drow=drow):
                rdma = pltpu.make_async_remote_copy(
                    src_ref=y_ref.at[pl.ds(idx, 1)],
                    dst_ref=out_ref.at[pl.ds(drow, 1)],
                    send_sem=send_sems.at[k],
                    recv_sem=recv_sem,
                    device_id=(dest,),
                    device_id_type=pl.DeviceIdType.MESH,
                )
                rdma.start()

            @pl.when(local)
            def _(idx=idx, drow=drow):
                out_ref[pl.ds(drow, 1), :] = y_ref[pl.ds(idx, 1), :]

        expected = jnp.int32(0)
        base = me * ROWS_PER_DEV
        for j in range(ROWS_PER_DEV):
            sel = row_iota == base + j
            kj = jnp.max((sel & keep).astype(jnp.int32)) > 0
            ej = jnp.max(jnp.where(sel, e, 0))
            owner = ej // EXP_PER_DEV
            expected = expected + (kj & (owner != me)).astype(jnp.int32)

            @pl.when(jnp.logical_not(kj))
            def _(j=j):
                out_ref[pl.ds(j, 1), :] = jnp.zeros((1, D_OUT), jnp.float32)

        def wait_one(i, carry):
            dummy = pltpu.make_async_remote_copy(
                src_ref=y_ref.at[pl.ds(0, 1)],
                dst_ref=out_ref.at[pl.ds(0, 1)],
                send_sem=send_sems.at[0],
                recv_sem=recv_sem,
                device_id=(me,),
                device_id_type=pl.DeviceIdType.MESH,
            )
            dummy.wait_recv()
            return carry

        lax.fori_loop(0, expected, wait_one, jnp.int32(0))

        for k in range(SLOTS):
            e_loc, s = divmod(k, CAP)
            idx = e_loc * PAD + s

            @pl.when(remote_flags[k])
            def _(idx=idx, k=k):
                d = pltpu.make_async_remote_copy(
                    src_ref=y_ref.at[pl.ds(idx, 1)],
                    dst_ref=out_ref.at[pl.ds(0, 1)],
                    send_sem=send_sems.at[k],
                    recv_sem=recv_sem,
                    device_id=(me,),
                    device_id_type=pl.DeviceIdType.MESH,
                )
                d.wait_send()

    return pl.pallas_call(
        body,
        out_shape=jax.ShapeDtypeStruct((ROWS_PER_DEV, D_OUT), jnp.float32),
        in_specs=[
            pl.BlockSpec(memory_space=pltpu.VMEM),
            pl.BlockSpec(memory_space=pltpu.VMEM),
            pl.BlockSpec(memory_space=pltpu.VMEM),
        ],
        out_specs=pl.BlockSpec(memory_space=pltpu.VMEM),
        scratch_shapes=[
            pltpu.VMEM((EXP_PER_DEV * PAD, D_IN), jnp.float32),
            pltpu.VMEM((EXP_PER_DEV * PAD, D_OUT), jnp.float32),
            pltpu.SemaphoreType.DMA((SLOTS,)),
            pltpu.SemaphoreType.DMA,
        ],
    )(x, route_idx, expert_W)


# device time: 22459 ns/iter; 1.0722x vs baseline; 1.0722x over previous
import jax
import jax.numpy as jnp
from jax import lax
from jax.experimental import pallas as pl
from jax.experimental.pallas import tpu as pltpu

N_DEV = 32
N_ROWS = 512
D_IN = 256
D_OUT = 512
N_EXP = 64
CAP = 6
EXP_PER_DEV = 2
ROWS_PER_DEV = N_ROWS // N_DEV
SLOTS = EXP_PER_DEV * CAP
PAD = 8

_CONTRACT0 = (((0,), (0,)), ((), ()))


def kernel(x, router_W, route_idx, expert_W):
    del router_W

    def body(x_ref, idx_ref, w_ref, out_ref, y_ref, send_sems, recv_sem):
        me = lax.axis_index("i")
        me_f = me.astype(jnp.float32)

        e = idx_ref[:, :]
        row_iota = lax.broadcasted_iota(jnp.int32, (N_ROWS, 1), 0)
        row_iota_f = row_iota.astype(jnp.float32)

        col = lax.broadcasted_iota(jnp.int32, (N_ROWS, N_EXP), 1)
        onehot = (col == e).astype(jnp.float32)
        tri_r = lax.broadcasted_iota(jnp.int32, (N_ROWS, N_ROWS), 0)
        tri_c = lax.broadcasted_iota(jnp.int32, (N_ROWS, N_ROWS), 1)
        tri = (tri_c <= tri_r).astype(jnp.float32)
        counts = jnp.dot(tri, onehot, preferred_element_type=jnp.float32)
        inc = jnp.sum(counts * onehot, axis=1, keepdims=True)
        keep = inc <= float(CAP)

        g = jnp.where(keep, e * CAP + inc.astype(jnp.int32) - 1, -1)

        col16 = lax.broadcasted_iota(jnp.int32, (N_ROWS, EXP_PER_DEV * PAD), 1)
        e_loc_col = col16 // PAD
        s_col = col16 % PAD
        tgt = SLOTS * me + CAP * e_loc_col + s_col
        sel16 = ((g == tgt) & (s_col < CAP)).astype(jnp.float32)

        meta_c = jnp.concatenate(
            [
                (row_iota // ROWS_PER_DEV).astype(jnp.float32),
                lax.rem(row_iota, ROWS_PER_DEV).astype(jnp.float32),
                jnp.ones((N_ROWS, 1), jnp.float32),
            ],
            axis=1,
        )
        slot_tbl = lax.dot_general(
            sel16, meta_c, _CONTRACT0, preferred_element_type=jnp.float32
        )

        xg = lax.dot_general(
            sel16,
            x_ref[:, :],
            _CONTRACT0,
            precision=lax.Precision.HIGHEST,
            preferred_element_type=jnp.float32,
        )
        for e_loc in range(EXP_PER_DEV):
            y_ref[pl.ds(e_loc * PAD, PAD), :] = jnp.dot(
                xg[e_loc * PAD : (e_loc + 1) * PAD, :],
                w_ref[e_loc, :, :],
                preferred_element_type=jnp.float32,
            )

        remote_flags = []
        for k in range(SLOTS):
            e_loc, s = divmod(k, CAP)
            idx = e_loc * PAD + s
            valid = slot_tbl[idx, 2] > 0.5
            dest = slot_tbl[idx, 0].astype(jnp.int32)
            drow = slot_tbl[idx, 1].astype(jnp.int32)
            remote = valid & (dest != me)
            local = valid & (dest == me)
            remote_flags.append(remote)

            @pl.when(remote)
            def _(idx=idx, k=k, dest=dest, drow=drow):
                rdma = pltpu.make_async_remote_copy(
                    src_ref=y_ref.at[pl.ds(idx, 1)],
                    dst_ref=out_ref.at[pl.ds(drow, 1)],
                    send_sem=send_sems.at[k],
                    recv_sem=recv_sem,
                    device_id=(dest,),
                    device_id_type=pl.DeviceIdType.MESH,
                )
                rdma.start()

            @pl.when(local)
            def _(idx=idx, drow=drow):
                out_ref[pl.ds(drow, 1), :] = y_ref[pl.ds(idx, 1), :]

        base = me * ROWS_PER_DEV
        colj = lax.broadcasted_iota(jnp.int32, (N_ROWS, ROWS_PER_DEV), 1)
        sel_out = (row_iota == base + colj).astype(jnp.float32)
        recv_c = jnp.concatenate(
            [keep.astype(jnp.float32), (e // EXP_PER_DEV).astype(jnp.float32)],
            axis=1,
        )
        m = lax.dot_general(
            sel_out, recv_c, _CONTRACT0, preferred_element_type=jnp.float32
        )
        expected = jnp.sum(
            ((m[:, 0:1] > 0.5) & (m[:, 1:2] != me_f)).astype(jnp.int32)
        )

        for j in range(ROWS_PER_DEV):
            @pl.when(m[j, 0] < 0.5)
            def _(j=j):
                out_ref[pl.ds(j, 1), :] = jnp.zeros((1, D_OUT), jnp.float32)

        def wait_one(i, carry):
            dummy = pltpu.make_async_remote_copy(
                src_ref=y_ref.at[pl.ds(0, 1)],
                dst_ref=out_ref.at[pl.ds(0, 1)],
                send_sem=send_sems.at[0],
                recv_sem=recv_sem,
                device_id=(me,),
                device_id_type=pl.DeviceIdType.MESH,
            )
            dummy.wait_recv()
            return carry

        lax.fori_loop(0, expected, wait_one, jnp.int32(0))

        for k in range(SLOTS):
            e_loc, s = divmod(k, CAP)
            idx = e_loc * PAD + s

            @pl.when(remote_flags[k])
            def _(idx=idx, k=k):
                d = pltpu.make_async_remote_copy(
                    src_ref=y_ref.at[pl.ds(idx, 1)],
                    dst_ref=out_ref.at[pl.ds(0, 1)],
                    send_sem=send_sems.at[k],
                    recv_sem=recv_sem,
                    device_id=(me,),
                    device_id_type=pl.DeviceIdType.MESH,
                )
                d.wait_send()

    return pl.pallas_call(
        body,
        out_shape=jax.ShapeDtypeStruct((ROWS_PER_DEV, D_OUT), jnp.float32),
        in_specs=[
            pl.BlockSpec(memory_space=pltpu.VMEM),
            pl.BlockSpec(memory_space=pltpu.VMEM),
            pl.BlockSpec(memory_space=pltpu.VMEM),
        ],
        out_specs=pl.BlockSpec(memory_space=pltpu.VMEM),
        scratch_shapes=[
            pltpu.VMEM((EXP_PER_DEV * PAD, D_OUT), jnp.float32),
            pltpu.SemaphoreType.DMA((SLOTS,)),
            pltpu.SemaphoreType.DMA,
        ],
    )(x, route_idx, expert_W)


# device time: 15087 ns/iter; 1.5961x vs baseline; 1.4886x over previous
import jax
import jax.numpy as jnp
from jax import lax
from jax.experimental import pallas as pl
from jax.experimental.pallas import tpu as pltpu

N_DEV = 32
N_ROWS = 512
D_IN = 256
D_OUT = 512
N_EXP = 64
CAP = 6
EXP_PER_DEV = 2
ROWS_PER_DEV = N_ROWS // N_DEV
SLOTS = EXP_PER_DEV * CAP
PAD = 8

_CONTRACT0 = (((0,), (0,)), ((), ()))


def kernel(x, router_W, route_idx, expert_W):
    del router_W

    def body(x_ref, idx_ref, w_ref, out_ref, y_ref, send_sems, recv_sem):
        me = lax.axis_index("i")
        me_f = me.astype(jnp.float32)

        barrier_sem = pltpu.get_barrier_semaphore()
        for dev in range(N_DEV):
            pl.semaphore_signal(
                barrier_sem,
                inc=1,
                device_id=(dev,),
                device_id_type=pl.DeviceIdType.MESH,
            )

        e = idx_ref[:, :]
        row_iota = lax.broadcasted_iota(jnp.int32, (N_ROWS, 1), 0)
        row_iota_f = row_iota.astype(jnp.float32)

        col = lax.broadcasted_iota(jnp.int32, (N_ROWS, N_EXP), 1)
        onehot = (col == e).astype(jnp.float32)
        tri_r = lax.broadcasted_iota(jnp.int32, (N_ROWS, N_ROWS), 0)
        tri_c = lax.broadcasted_iota(jnp.int32, (N_ROWS, N_ROWS), 1)
        tri = (tri_c <= tri_r).astype(jnp.float32)
        counts = jnp.dot(tri, onehot, preferred_element_type=jnp.float32)
        inc = jnp.sum(counts * onehot, axis=1, keepdims=True)
        keep = inc <= float(CAP)

        g = jnp.where(keep, e * CAP + inc.astype(jnp.int32) - 1, -1)

        col16 = lax.broadcasted_iota(jnp.int32, (N_ROWS, EXP_PER_DEV * PAD), 1)
        e_loc_col = col16 // PAD
        s_col = col16 % PAD
        tgt = SLOTS * me + CAP * e_loc_col + s_col
        sel16 = ((g == tgt) & (s_col < CAP)).astype(jnp.float32)

        meta_c = jnp.concatenate(
            [
                (row_iota // ROWS_PER_DEV).astype(jnp.float32),
                lax.rem(row_iota, ROWS_PER_DEV).astype(jnp.float32),
                jnp.ones((N_ROWS, 1), jnp.float32),
            ],
            axis=1,
        )
        slot_tbl = lax.dot_general(
            sel16, meta_c, _CONTRACT0, preferred_element_type=jnp.float32
        )

        xg = lax.dot_general(
            sel16,
            x_ref[:, :],
            _CONTRACT0,
            precision=lax.Precision.HIGHEST,
            preferred_element_type=jnp.float32,
        )
        for e_loc in range(EXP_PER_DEV):
            y_ref[pl.ds(e_loc * PAD, PAD), :] = jnp.dot(
                xg[e_loc * PAD : (e_loc + 1) * PAD, :],
                w_ref[e_loc, :, :],
                preferred_element_type=jnp.float32,
            )

        pl.semaphore_wait(barrier_sem, N_DEV)

        remote_flags = []
        for k in range(SLOTS):
            e_loc, s = divmod(k, CAP)
            idx = e_loc * PAD + s
            valid = slot_tbl[idx, 2] > 0.5
            dest = slot_tbl[idx, 0].astype(jnp.int32)
            drow = slot_tbl[idx, 1].astype(jnp.int32)
            remote = valid & (dest != me)
            local = valid & (dest == me)
            remote_flags.append(remote)

            @pl.when(remote)
            def _(idx=idx, k=k, dest=dest, drow=drow):
                rdma = pltpu.make_async_remote_copy(
                    src_ref=y_ref.at[pl.ds(idx, 1)],
                    dst_ref=out_ref.at[pl.ds(drow, 1)],
                    send_sem=send_sems.at[k],
                    recv_sem=recv_sem,
                    device_id=(dest,),
                    device_id_type=pl.DeviceIdType.MESH,
                )
                rdma.start()

            @pl.when(local)
            def _(idx=idx, drow=drow):
                out_ref[pl.ds(drow, 1), :] = y_ref[pl.ds(idx, 1), :]

        base = me * ROWS_PER_DEV
        colj = lax.broadcasted_iota(jnp.int32, (N_ROWS, ROWS_PER_DEV), 1)
        sel_out = (row_iota == base + colj).astype(jnp.float32)
        recv_c = jnp.concatenate(
            [keep.astype(jnp.float32), (e // EXP_PER_DEV).astype(jnp.float32)],
            axis=1,
        )
        m = lax.dot_general(
            sel_out, recv_c, _CONTRACT0, preferred_element_type=jnp.float32
        )
        expected = jnp.sum(
            ((m[:, 0:1] > 0.5) & (m[:, 1:2] != me_f)).astype(jnp.int32)
        )

        for j in range(ROWS_PER_DEV):
            @pl.when(m[j, 0] < 0.5)
            def _(j=j):
                out_ref[pl.ds(j, 1), :] = jnp.zeros((1, D_OUT), jnp.float32)

        def wait_one(i, carry):
            dummy = pltpu.make_async_remote_copy(
                src_ref=y_ref.at[pl.ds(0, 1)],
                dst_ref=out_ref.at[pl.ds(0, 1)],
                send_sem=send_sems.at[0],
                recv_sem=recv_sem,
                device_id=(me,),
                device_id_type=pl.DeviceIdType.MESH,
            )
            dummy.wait_recv()
            return carry

        lax.fori_loop(0, expected, wait_one, jnp.int32(0))

        for k in range(SLOTS):
            e_loc, s = divmod(k, CAP)
            idx = e_loc * PAD + s

            @pl.when(remote_flags[k])
            def _(idx=idx, k=k):
                d = pltpu.make_async_remote_copy(
                    src_ref=y_ref.at[pl.ds(idx, 1)],
                    dst_ref=out_ref.at[pl.ds(0, 1)],
                    send_sem=send_sems.at[k],
                    recv_sem=recv_sem,
                    device_id=(me,),
                    device_id_type=pl.DeviceIdType.MESH,
                )
                d.wait_send()

    return pl.pallas_call(
        body,
        out_shape=jax.ShapeDtypeStruct((ROWS_PER_DEV, D_OUT), jnp.float32),
        in_specs=[
            pl.BlockSpec(memory_space=pltpu.VMEM),
            pl.BlockSpec(memory_space=pltpu.VMEM),
            pl.BlockSpec(memory_space=pltpu.VMEM),
        ],
        out_specs=pl.BlockSpec(memory_space=pltpu.VMEM),
        scratch_shapes=[
            pltpu.VMEM((EXP_PER_DEV * PAD, D_OUT), jnp.float32),
            pltpu.SemaphoreType.DMA((SLOTS,)),
            pltpu.SemaphoreType.DMA,
        ],
        compiler_params=pltpu.CompilerParams(collective_id=0),
    )(x, route_idx, expert_W)
